# baseline (speedup 1.0000x reference)
import jax
import jax.numpy as jnp
from jax.experimental import pallas as pl
from jax.experimental.pallas import tpu as pltpu

_BN = 512


def _make_body(batch):
    def _proj_body(p_ref, x_ref, o_ref, xs_ref):
        i = pl.program_id(0)
        k = pl.program_id(1)

        @pl.when(jnp.logical_and(i == 0, k == 0))
        def _cache_first_half():
            xs_ref[...] = x_ref[...]

        @pl.when(k == 0)
        def _first_half():
            p = p_ref[...]
            for b in range(batch):
                o_ref[b] = jnp.dot(p, xs_ref[b], preferred_element_type=jnp.float32)

        @pl.when(k == 1)
        def _second_half():
            p = p_ref[...]
            for b in range(batch):
                o_ref[b] = o_ref[b] + jnp.dot(
                    p, x_ref[b], preferred_element_type=jnp.float32
                )

    return _proj_body


def kernel(x, projection_matrix):
    B, N, D = x.shape
    half = N // 2
    grid = (N // _BN, 2)
    return pl.pallas_call(
        _make_body(B),
        grid=grid,
        in_specs=[
            pl.BlockSpec((_BN, half), lambda i, k: (i, k)),
            pl.BlockSpec(
                (B, half, D),
                lambda i, k: (0, jnp.where(jnp.logical_and(i == 0, k == 0), 0, 1), 0),
            ),
        ],
        out_specs=pl.BlockSpec((B, _BN, D), lambda i, k: (0, i, 0)),
        out_shape=jax.ShapeDtypeStruct((B, N, D), jnp.float32),
        scratch_shapes=[pltpu.VMEM((B, half, D), jnp.float32)],
        compiler_params=pltpu.CompilerParams(
            dimension_semantics=("parallel", "arbitrary"),
        ),
    )(projection_matrix, x)


# speedup vs baseline: 1.0957x; 1.0957x over previous
import jax
import jax.numpy as jnp
from jax.experimental import pallas as pl
from jax.experimental.pallas import tpu as pltpu

_BN = 512


def _make_body(batch):
    def _proj_body(p_ref, x_ref, o_ref):
        p = p_ref[...]
        for b in range(batch):
            o_ref[b] = jnp.dot(p, x_ref[b], preferred_element_type=jnp.float32)

    return _proj_body


def kernel(x, projection_matrix):
    B, N, D = x.shape
    grid = (N // _BN,)
    return pl.pallas_call(
        _make_body(B),
        grid=grid,
        in_specs=[
            pl.BlockSpec((_BN, N), lambda i: (i, 0)),
            pl.BlockSpec((B, N, D), lambda i: (0, 0, 0)),
        ],
        out_specs=pl.BlockSpec((B, _BN, D), lambda i: (0, i, 0)),
        out_shape=jax.ShapeDtypeStruct((B, N, D), jnp.float32),
        compiler_params=pltpu.CompilerParams(
            dimension_semantics=("parallel",),
        ),
    )(projection_matrix, x)
